# baseline (speedup 1.0000x reference)
import functools

import jax
import jax.numpy as jnp
from jax import lax
from jax.experimental import pallas as pl
from jax.experimental.pallas import tpu as pltpu
from jax.experimental.pallas import tpu_sc as plsc

N_NODES = 10000
N_EDGES = 160000
D = 256
DH = 128
NC = 2
NS = 16
CHUNK = 96
CH = 108
NPASS = 4
HC = CH // NPASS
NBUF = 3
EDGES_PER_TILE = CH * CHUNK
EDGES_PAD = NS * EDGES_PER_TILE
ACC_R = 10112
ROWS_PER_TILE_ZERO = ACC_R // NS
OUT_ROWS_PER_TILE = 632
OUT_R = NS * OUT_ROWS_PER_TILE


def _sc_body(xc_hbm, srci_hbm, dsti_hbm, zeros_hbm, out_hbm,
             acc, src_v, dst_v, gbuf, gsem, ssem):
    c = lax.axis_index("c")
    s = lax.axis_index("s")
    pltpu.sync_copy(zeros_hbm,
                    acc.at[pl.ds(s * ROWS_PER_TILE_ZERO, ROWS_PER_TILE_ZERO)])
    plsc.subcore_barrier()

    def pass_body(p, carry):
        pltpu.sync_copy(srci_hbm.at[c, s, p], src_v)
        pltpu.sync_copy(dsti_hbm.at[s, p], dst_v)
        pltpu.async_copy(xc_hbm.at[src_v.at[0]], gbuf.at[0], gsem)

        def body(g, carry2):
            for b in range(NBUF):
                j = NBUF * g + b
                @pl.when(j >= 2)
                def _():
                    pltpu.make_async_copy(gbuf.at[(b + 1) % NBUF],
                                          acc.at[dst_v.at[0]], ssem).wait()

                @pl.when(j + 1 < HC)
                def _():
                    pltpu.async_copy(xc_hbm.at[src_v.at[j + 1]],
                                     gbuf.at[(b + 1) % NBUF], gsem)

                pltpu.make_async_copy(xc_hbm.at[src_v.at[0]], gbuf.at[b],
                                      gsem).wait()
                pltpu.async_copy(gbuf.at[b], acc.at[dst_v.at[j]], ssem, add=True)
            return carry2

        lax.fori_loop(0, HC // NBUF, body, 0)
        pltpu.make_async_copy(gbuf.at[1], acc.at[dst_v.at[0]], ssem).wait()
        pltpu.make_async_copy(gbuf.at[2], acc.at[dst_v.at[0]], ssem).wait()
        return carry

    lax.fori_loop(0, NPASS, pass_body, 0)
    plsc.subcore_barrier()
    pltpu.sync_copy(acc.at[pl.ds(s * OUT_ROWS_PER_TILE, OUT_ROWS_PER_TILE)],
                    out_hbm.at[c, pl.ds(s * OUT_ROWS_PER_TILE, OUT_ROWS_PER_TILE)])


def _segment_sums(x_cols, src_r, dst_r, zeros):
    mesh = plsc.VectorSubcoreMesh(core_axis_name="c", subcore_axis_name="s")
    return pl.kernel(
        _sc_body,
        out_type=jax.ShapeDtypeStruct((NC, OUT_R, DH), jnp.float32),
        mesh=mesh,
        scratch_types=[
            pltpu.VMEM_SHARED((ACC_R, DH), jnp.float32),
            pltpu.VMEM((HC, CHUNK), jnp.int32),
            pltpu.VMEM((HC, CHUNK), jnp.int32),
            pltpu.VMEM((NBUF, CHUNK, DH), jnp.float32),
            pltpu.SemaphoreType.DMA,
            pltpu.SemaphoreType.DMA,
        ],
    )(x_cols, src_r, dst_r, zeros)


def _tc_body(x_ref, s0_ref, s1_ref, w1_ref, b1_ref, w2_ref, b2_ref,
             g_ref, bt_ref, o_ref):
    sums = jnp.concatenate([s0_ref[...], s1_ref[...]], axis=-1)
    nrm = jnp.sqrt(jnp.sum(sums * sums, axis=-1, keepdims=True))
    agg = sums / (nrm + 1e-8)
    h = x_ref[...] + agg
    h = jnp.maximum(
        jnp.dot(h, w1_ref[...], preferred_element_type=jnp.float32) + b1_ref[...],
        0.0)
    h = jnp.dot(h, w2_ref[...], preferred_element_type=jnp.float32) + b2_ref[...]
    mu = jnp.mean(h, axis=-1, keepdims=True)
    var = jnp.mean((h - mu) * (h - mu), axis=-1, keepdims=True)
    o_ref[...] = (h - mu) / jnp.sqrt(var + 1e-5) * g_ref[...] + bt_ref[...]


def _dense(x, sums, W1, b1, W2, b2, gamma, beta):
    R = 1000
    return pl.pallas_call(
        _tc_body,
        grid=(N_NODES // R,),
        in_specs=[
            pl.BlockSpec((R, D), lambda i: (i, 0)),
            pl.BlockSpec((R, DH), lambda i: (i, 0)),
            pl.BlockSpec((R, DH), lambda i: (i, 0)),
            pl.BlockSpec((D, D), lambda i: (0, 0)),
            pl.BlockSpec((1, D), lambda i: (0, 0)),
            pl.BlockSpec((D, D), lambda i: (0, 0)),
            pl.BlockSpec((1, D), lambda i: (0, 0)),
            pl.BlockSpec((1, D), lambda i: (0, 0)),
            pl.BlockSpec((1, D), lambda i: (0, 0)),
        ],
        out_specs=pl.BlockSpec((R, D), lambda i: (i, 0)),
        out_shape=jax.ShapeDtypeStruct((N_NODES, D), jnp.float32),
    )(x, sums[0], sums[1], W1, b1.reshape(1, D), W2, b2.reshape(1, D),
      gamma.reshape(1, D), beta.reshape(1, D))


@jax.jit
def kernel(x, edge_index, W1, b1, W2, b2, gamma, beta):
    src = edge_index[0]
    dst = edge_index[1]
    npad = EDGES_PAD - N_EDGES
    pad_i = jnp.arange(npad, dtype=jnp.int32)
    src_p = jnp.concatenate([src, (pad_i * 37) % N_NODES])
    dst_p = jnp.concatenate([dst, N_NODES + pad_i % (ACC_R - N_NODES)])
    src_r = jnp.stack([2 * src_p, 2 * src_p + 1]).reshape(NC, NS, NPASS, HC, CHUNK)
    dst_r = dst_p.reshape(NS, NPASS, HC, CHUNK)
    x_cols = x.reshape(2 * N_NODES, DH)
    zeros = jnp.zeros((ROWS_PER_TILE_ZERO, DH), jnp.float32)
    sums = _segment_sums(x_cols, src_r, dst_r, zeros)
    return _dense(x, sums, W1, b1, W2, b2, gamma, beta)


# speedup vs baseline: 1.0176x; 1.0176x over previous
import functools

import jax
import jax.numpy as jnp
from jax import lax
from jax.experimental import pallas as pl
from jax.experimental.pallas import tpu as pltpu
from jax.experimental.pallas import tpu_sc as plsc

N_NODES = 10000
N_EDGES = 160000
D = 256
DH = 128
NC = 2
NS = 16
CHUNK = 128
CH = 80
NPASS = 2
HC = CH // NPASS
EDGES_PER_TILE = CH * CHUNK
EDGES_PAD = NS * EDGES_PER_TILE
ACC_R = 10112
ROWS_PER_TILE_ZERO = ACC_R // NS
OUT_ROWS_PER_TILE = 632
OUT_R = NS * OUT_ROWS_PER_TILE


def _sc_body(xc_hbm, srci_hbm, dsti_hbm, zeros_hbm, out_hbm,
             acc, src_v, dst_v, gbuf, gsem, ssem):
    c = lax.axis_index("c")
    s = lax.axis_index("s")
    pltpu.sync_copy(zeros_hbm,
                    acc.at[pl.ds(s * ROWS_PER_TILE_ZERO, ROWS_PER_TILE_ZERO)])
    plsc.subcore_barrier()

    def pass_body(p, carry):
        pltpu.sync_copy(srci_hbm.at[c, s, pl.ds(p * HC, HC)], src_v)
        pltpu.sync_copy(dsti_hbm.at[s, pl.ds(p * HC, HC)], dst_v)
        pltpu.async_copy(xc_hbm.at[src_v.at[0]], gbuf.at[0], gsem)

        def body(g, carry2):
            for b in range(2):
                j = 2 * g + b
                @pl.when(j >= 1)
                def _():
                    pltpu.make_async_copy(gbuf.at[1 - b], acc.at[dst_v.at[0]],
                                          ssem).wait()

                @pl.when(j + 1 < HC)
                def _():
                    pltpu.async_copy(xc_hbm.at[src_v.at[j + 1]], gbuf.at[1 - b],
                                     gsem)

                pltpu.make_async_copy(xc_hbm.at[src_v.at[0]], gbuf.at[b],
                                      gsem).wait()
                pltpu.async_copy(gbuf.at[b], acc.at[dst_v.at[j]], ssem, add=True)
            return carry2

        lax.fori_loop(0, HC // 2, body, 0)
        pltpu.make_async_copy(gbuf.at[1], acc.at[dst_v.at[0]], ssem).wait()
        return carry

    lax.fori_loop(0, NPASS, pass_body, 0)
    plsc.subcore_barrier()
    pltpu.sync_copy(acc.at[pl.ds(s * OUT_ROWS_PER_TILE, OUT_ROWS_PER_TILE)],
                    out_hbm.at[c, pl.ds(s * OUT_ROWS_PER_TILE, OUT_ROWS_PER_TILE)])


def _segment_sums(x_cols, src_r, dst_r, zeros):
    mesh = plsc.VectorSubcoreMesh(core_axis_name="c", subcore_axis_name="s")
    return pl.kernel(
        _sc_body,
        out_type=jax.ShapeDtypeStruct((NC, OUT_R, DH), jnp.float32),
        mesh=mesh,
        scratch_types=[
            pltpu.VMEM_SHARED((ACC_R, DH), jnp.float32),
            pltpu.VMEM((HC, CHUNK), jnp.int32),
            pltpu.VMEM((HC, CHUNK), jnp.int32),
            pltpu.VMEM((2, CHUNK, DH), jnp.float32),
            pltpu.SemaphoreType.DMA,
            pltpu.SemaphoreType.DMA,
        ],
    )(x_cols, src_r, dst_r, zeros)


def _tc_body(x_ref, s0_ref, s1_ref, w1_ref, b1_ref, w2_ref, b2_ref,
             g_ref, bt_ref, o_ref):
    sums = jnp.concatenate([s0_ref[...], s1_ref[...]], axis=-1)
    nrm = jnp.sqrt(jnp.sum(sums * sums, axis=-1, keepdims=True))
    agg = sums / (nrm + 1e-8)
    h = x_ref[...] + agg
    h = jnp.maximum(
        jnp.dot(h, w1_ref[...], preferred_element_type=jnp.float32) + b1_ref[...],
        0.0)
    h = jnp.dot(h, w2_ref[...], preferred_element_type=jnp.float32) + b2_ref[...]
    mu = jnp.mean(h, axis=-1, keepdims=True)
    var = jnp.mean((h - mu) * (h - mu), axis=-1, keepdims=True)
    o_ref[...] = (h - mu) / jnp.sqrt(var + 1e-5) * g_ref[...] + bt_ref[...]


def _dense(x, sums, W1, b1, W2, b2, gamma, beta):
    R = 2000
    return pl.pallas_call(
        _tc_body,
        grid=(N_NODES // R,),
        in_specs=[
            pl.BlockSpec((R, D), lambda i: (i, 0)),
            pl.BlockSpec((R, DH), lambda i: (i, 0)),
            pl.BlockSpec((R, DH), lambda i: (i, 0)),
            pl.BlockSpec((D, D), lambda i: (0, 0)),
            pl.BlockSpec((1, D), lambda i: (0, 0)),
            pl.BlockSpec((D, D), lambda i: (0, 0)),
            pl.BlockSpec((1, D), lambda i: (0, 0)),
            pl.BlockSpec((1, D), lambda i: (0, 0)),
            pl.BlockSpec((1, D), lambda i: (0, 0)),
        ],
        out_specs=pl.BlockSpec((R, D), lambda i: (i, 0)),
        out_shape=jax.ShapeDtypeStruct((N_NODES, D), jnp.float32),
    )(x, sums[0], sums[1], W1, b1.reshape(1, D), W2, b2.reshape(1, D),
      gamma.reshape(1, D), beta.reshape(1, D))


@jax.jit
def kernel(x, edge_index, W1, b1, W2, b2, gamma, beta):
    src = edge_index[0]
    dst = edge_index[1]
    npad = EDGES_PAD - N_EDGES
    pad_i = jnp.arange(npad, dtype=jnp.int32)
    src_p = jnp.concatenate([src, (pad_i * 37) % N_NODES])
    dst_p = jnp.concatenate([dst, N_NODES + pad_i % (ACC_R - N_NODES)])
    src_r = jnp.stack([2 * src_p, 2 * src_p + 1]).reshape(NC, NS, CH, CHUNK)
    dst_r = dst_p.reshape(NS, CH, CHUNK)
    x_cols = x.reshape(2 * N_NODES, DH)
    zeros = jnp.zeros((ROWS_PER_TILE_ZERO, DH), jnp.float32)
    sums = _segment_sums(x_cols, src_r, dst_r, zeros)
    return _dense(x, sums, W1, b1, W2, b2, gamma, beta)
